# baseline (speedup 1.0000x reference)
import functools

import jax
import jax.numpy as jnp
from jax import lax
from jax.experimental import pallas as pl
from jax.experimental.pallas import tpu as pltpu
from jax.experimental.pallas import tpu_sc as plsc

_LEVELS = 7
_ROWS, _COLS = 32768, 768

_NW = 32
_W_ROWS = _ROWS // _NW
_RCH = 16
_NBUF = 2
_NCH = _W_ROWS // _RCH
_ROUNDS = _NCH // _NBUF
_VECS = _COLS // 16


def _sc_body(z_hbm, sc_hbm, sh_hbm, st_hbm, b0_hbm, idx_hbm, zq_hbm,
             zbuf, ibuf, qbuf, svec, hvec, tvec, bvec, sem_in, sem_out):
    wid = lax.axis_index("s") * 2 + lax.axis_index("c")
    base = wid * _W_ROWS
    pltpu.sync_copy(sc_hbm, svec)
    pltpu.sync_copy(sh_hbm, hvec)
    pltpu.sync_copy(st_hbm, tvec)
    pltpu.sync_copy(b0_hbm, bvec)
    scale = svec[...]
    shift = hvec[...]
    stepv = tvec[...]
    b0v = bvec[...]
    lo = jnp.full((16,), 0.5, jnp.float32)
    hi = jnp.full((16,), _LEVELS - 0.25, jnp.float32)

    def compute_chunk(b):
        def row_step(rr, _):
            @plsc.parallel_loop(0, _VECS, 1, unroll=8)
            def _vec_step(i):
                zv = zbuf[b, rr, pl.ds(i * 16, 16)]
                t = jnp.minimum(jnp.maximum(zv * scale + shift, lo), hi)
                iv = t.astype(jnp.int32)
                ibuf[b, rr, pl.ds(i * 16, 16)] = iv
                qbuf[b, rr, pl.ds(i * 16, 16)] = (
                    iv.astype(jnp.float32) * stepv + b0v)
            return 0
        lax.fori_loop(0, _RCH, row_step, 0)

    for b in range(_NBUF):
        pltpu.async_copy(z_hbm.at[pl.ds(base + b * _RCH, _RCH)],
                         zbuf.at[b], sem_in)

    def round_step(gr, _):
        for b in range(_NBUF):
            g = gr * _NBUF + b
            row0 = base + g * _RCH
            pltpu.make_async_copy(z_hbm.at[pl.ds(0, _RCH)], zbuf.at[b],
                                  sem_in).wait()

            @pl.when(gr > 0)
            def _wait_out():
                pltpu.make_async_copy(ibuf.at[b],
                                      idx_hbm.at[pl.ds(0, _RCH)],
                                      sem_out).wait()
                pltpu.make_async_copy(qbuf.at[b],
                                      zq_hbm.at[pl.ds(0, _RCH)],
                                      sem_out).wait()

            compute_chunk(b)
            pltpu.async_copy(ibuf.at[b], idx_hbm.at[pl.ds(row0, _RCH)],
                             sem_out)
            pltpu.async_copy(qbuf.at[b], zq_hbm.at[pl.ds(row0, _RCH)],
                             sem_out)

            @pl.when(gr < _ROUNDS - 1)
            def _next_in():
                pltpu.async_copy(
                    z_hbm.at[pl.ds(row0 + _NBUF * _RCH, _RCH)],
                    zbuf.at[b], sem_in)
        return 0

    lax.fori_loop(0, _ROUNDS, round_step, 0)
    for b in range(_NBUF):
        pltpu.make_async_copy(ibuf.at[b], idx_hbm.at[pl.ds(0, _RCH)],
                              sem_out).wait()
        pltpu.make_async_copy(qbuf.at[b], zq_hbm.at[pl.ds(0, _RCH)],
                              sem_out).wait()


def _sc_quantize(z, scale16, shift16, step16, b016):
    mesh = plsc.VectorSubcoreMesh(core_axis_name="c", subcore_axis_name="s")
    call = functools.partial(
        pl.kernel,
        out_type=[
            jax.ShapeDtypeStruct((_ROWS, _COLS), jnp.int32),
            jax.ShapeDtypeStruct((_ROWS, _COLS), jnp.float32),
        ],
        mesh=mesh,
        scratch_types=[
            pltpu.VMEM((_NBUF, _RCH, _COLS), jnp.float32),
            pltpu.VMEM((_NBUF, _RCH, _COLS), jnp.int32),
            pltpu.VMEM((_NBUF, _RCH, _COLS), jnp.float32),
            pltpu.VMEM((16,), jnp.float32),
            pltpu.VMEM((16,), jnp.float32),
            pltpu.VMEM((16,), jnp.float32),
            pltpu.VMEM((16,), jnp.float32),
            pltpu.SemaphoreType.DMA,
            pltpu.SemaphoreType.DMA,
        ],
    )(_sc_body)
    return call(z, scale16, shift16, step16, b016)


def kernel(z, boundaries):
    b0 = boundaries[0]
    step = (boundaries[_LEVELS - 1] - b0) * (1.0 / (_LEVELS - 1))
    scale = 1.0 / step
    shift = -b0 * scale + 0.5
    scale16 = jnp.full((16,), scale, jnp.float32)
    shift16 = jnp.full((16,), shift, jnp.float32)
    step16 = jnp.full((16,), step, jnp.float32)
    b016 = jnp.full((16,), b0, jnp.float32)

    idx, zq = _sc_quantize(z, scale16, shift16, step16, b016)
    return zq, idx


# speedup vs baseline: 1.0004x; 1.0004x over previous
import functools

import jax
import jax.numpy as jnp
from jax import lax
from jax.experimental import pallas as pl
from jax.experimental.pallas import tpu as pltpu
from jax.experimental.pallas import tpu_sc as plsc

_LEVELS = 7
_ROWS, _COLS = 32768, 768

_NW = 32
_W_ROWS = _ROWS // _NW
_RCH = 8
_NBUF = 4
_NCH = _W_ROWS // _RCH
_ROUNDS = _NCH // _NBUF
_VECS = _COLS // 16


def _sc_body(z_hbm, sc_hbm, sh_hbm, st_hbm, b0_hbm, idx_hbm, zq_hbm,
             zbuf, ibuf, qbuf, svec, hvec, tvec, bvec, sem_in, sem_oi,
             sem_oq):
    wid = lax.axis_index("s") * 2 + lax.axis_index("c")
    base = wid * _W_ROWS
    pltpu.sync_copy(sc_hbm, svec)
    pltpu.sync_copy(sh_hbm, hvec)
    pltpu.sync_copy(st_hbm, tvec)
    pltpu.sync_copy(b0_hbm, bvec)
    scale = svec[...]
    shift = hvec[...]
    stepv = tvec[...]
    b0v = bvec[...]
    lo = jnp.full((16,), 0.5, jnp.float32)
    hi = jnp.full((16,), _LEVELS - 0.25, jnp.float32)

    def compute_chunk(b):
        def row_step(rr, _):
            @plsc.parallel_loop(0, _VECS, 1, unroll=16)
            def _vec_step(i):
                zv = zbuf[b, rr, pl.ds(i * 16, 16)]
                t = jnp.minimum(jnp.maximum(zv * scale + shift, lo), hi)
                iv = t.astype(jnp.int32)
                ibuf[b, rr, pl.ds(i * 16, 16)] = iv
                qbuf[b, rr, pl.ds(i * 16, 16)] = (
                    iv.astype(jnp.float32) * stepv + b0v)
            return 0
        lax.fori_loop(0, _RCH, row_step, 0)

    for b in range(_NBUF):
        pltpu.async_copy(z_hbm.at[pl.ds(base + b * _RCH, _RCH)],
                         zbuf.at[b], sem_in.at[b])

    def round_step(gr, _):
        for b in range(_NBUF):
            g = gr * _NBUF + b
            row0 = base + g * _RCH
            pltpu.make_async_copy(z_hbm.at[pl.ds(0, _RCH)], zbuf.at[b],
                                  sem_in.at[b]).wait()

            @pl.when(gr > 0)
            def _wait_out():
                pltpu.make_async_copy(ibuf.at[b],
                                      idx_hbm.at[pl.ds(0, _RCH)],
                                      sem_oi.at[b]).wait()
                pltpu.make_async_copy(qbuf.at[b],
                                      zq_hbm.at[pl.ds(0, _RCH)],
                                      sem_oq.at[b]).wait()

            compute_chunk(b)
            pltpu.async_copy(ibuf.at[b], idx_hbm.at[pl.ds(row0, _RCH)],
                             sem_oi.at[b])
            pltpu.async_copy(qbuf.at[b], zq_hbm.at[pl.ds(row0, _RCH)],
                             sem_oq.at[b])

            @pl.when(gr < _ROUNDS - 1)
            def _next_in():
                pltpu.async_copy(
                    z_hbm.at[pl.ds(row0 + _NBUF * _RCH, _RCH)],
                    zbuf.at[b], sem_in.at[b])
        return 0

    lax.fori_loop(0, _ROUNDS, round_step, 0)
    for b in range(_NBUF):
        pltpu.make_async_copy(ibuf.at[b], idx_hbm.at[pl.ds(0, _RCH)],
                              sem_oi.at[b]).wait()
        pltpu.make_async_copy(qbuf.at[b], zq_hbm.at[pl.ds(0, _RCH)],
                              sem_oq.at[b]).wait()


def _sc_quantize(z, scale16, shift16, step16, b016):
    mesh = plsc.VectorSubcoreMesh(core_axis_name="c", subcore_axis_name="s")
    call = functools.partial(
        pl.kernel,
        out_type=[
            jax.ShapeDtypeStruct((_ROWS, _COLS), jnp.int32),
            jax.ShapeDtypeStruct((_ROWS, _COLS), jnp.float32),
        ],
        mesh=mesh,
        scratch_types=[
            pltpu.VMEM((_NBUF, _RCH, _COLS), jnp.float32),
            pltpu.VMEM((_NBUF, _RCH, _COLS), jnp.int32),
            pltpu.VMEM((_NBUF, _RCH, _COLS), jnp.float32),
            pltpu.VMEM((16,), jnp.float32),
            pltpu.VMEM((16,), jnp.float32),
            pltpu.VMEM((16,), jnp.float32),
            pltpu.VMEM((16,), jnp.float32),
            pltpu.SemaphoreType.DMA((_NBUF,)),
            pltpu.SemaphoreType.DMA((_NBUF,)),
            pltpu.SemaphoreType.DMA((_NBUF,)),
        ],
    )(_sc_body)
    return call(z, scale16, shift16, step16, b016)


def kernel(z, boundaries):
    b0 = boundaries[0]
    step = (boundaries[_LEVELS - 1] - b0) * (1.0 / (_LEVELS - 1))
    scale = 1.0 / step
    shift = -b0 * scale + 0.5
    scale16 = jnp.full((16,), scale, jnp.float32)
    shift16 = jnp.full((16,), shift, jnp.float32)
    step16 = jnp.full((16,), step, jnp.float32)
    b016 = jnp.full((16,), b0, jnp.float32)

    idx, zq = _sc_quantize(z, scale16, shift16, step16, b016)
    return zq, idx


# speedup vs baseline: 1.0351x; 1.0347x over previous
import functools

import jax
import jax.numpy as jnp
from jax import lax
from jax.experimental import pallas as pl
from jax.experimental.pallas import tpu as pltpu
from jax.experimental.pallas import tpu_sc as plsc

_LEVELS = 7
_ROWS, _COLS = 32768, 768

_NW = 32
_W_ROWS = _ROWS // _NW
_RCH = 8
_NBUF = 4
_NCH = _W_ROWS // _RCH
_ROUNDS = _NCH // _NBUF
_VECS = _COLS // 16


def _sc_body(z_hbm, p_hbm, idx_hbm, zq_hbm,
             zbuf, ibuf, qbuf, pvec, sem_in, sem_oi, sem_oq):
    wid = lax.axis_index("s") * 2 + lax.axis_index("c")
    base = wid * _W_ROWS
    pltpu.sync_copy(p_hbm, pvec)
    scale = pvec[0]
    shift = pvec[1]
    stepv = pvec[2]
    b0v = pvec[3]
    lo = jnp.full((16,), 0.5, jnp.float32)
    hi = jnp.full((16,), _LEVELS - 0.25, jnp.float32)

    def compute_chunk(b):
        def row_step(rr, _):
            @plsc.parallel_loop(0, _VECS, 1, unroll=16)
            def _vec_step(i):
                zv = zbuf[b, rr, pl.ds(i * 16, 16)]
                t = jnp.minimum(jnp.maximum(zv * scale + shift, lo), hi)
                iv = t.astype(jnp.int32)
                ibuf[b, rr, pl.ds(i * 16, 16)] = iv
                qbuf[b, rr, pl.ds(i * 16, 16)] = (
                    iv.astype(jnp.float32) * stepv + b0v)
            return 0
        lax.fori_loop(0, _RCH, row_step, 0)

    for b in range(_NBUF):
        pltpu.async_copy(z_hbm.at[pl.ds(base + b * _RCH, _RCH)],
                         zbuf.at[b], sem_in.at[b])

    def round_step(gr, _):
        for b in range(_NBUF):
            g = gr * _NBUF + b
            row0 = base + g * _RCH
            pltpu.make_async_copy(z_hbm.at[pl.ds(0, _RCH)], zbuf.at[b],
                                  sem_in.at[b]).wait()

            @pl.when(gr > 0)
            def _wait_out():
                pltpu.make_async_copy(ibuf.at[b],
                                      idx_hbm.at[pl.ds(0, _RCH)],
                                      sem_oi.at[b]).wait()
                pltpu.make_async_copy(qbuf.at[b],
                                      zq_hbm.at[pl.ds(0, _RCH)],
                                      sem_oq.at[b]).wait()

            compute_chunk(b)
            pltpu.async_copy(ibuf.at[b], idx_hbm.at[pl.ds(row0, _RCH)],
                             sem_oi.at[b])
            pltpu.async_copy(qbuf.at[b], zq_hbm.at[pl.ds(row0, _RCH)],
                             sem_oq.at[b])

            @pl.when(gr < _ROUNDS - 1)
            def _next_in():
                pltpu.async_copy(
                    z_hbm.at[pl.ds(row0 + _NBUF * _RCH, _RCH)],
                    zbuf.at[b], sem_in.at[b])
        return 0

    lax.fori_loop(0, _ROUNDS, round_step, 0)
    for b in range(_NBUF):
        pltpu.make_async_copy(ibuf.at[b], idx_hbm.at[pl.ds(0, _RCH)],
                              sem_oi.at[b]).wait()
        pltpu.make_async_copy(qbuf.at[b], zq_hbm.at[pl.ds(0, _RCH)],
                              sem_oq.at[b]).wait()


def _sc_quantize(z, params):
    mesh = plsc.VectorSubcoreMesh(core_axis_name="c", subcore_axis_name="s")
    call = functools.partial(
        pl.kernel,
        out_type=[
            jax.ShapeDtypeStruct((_ROWS, _COLS), jnp.int32),
            jax.ShapeDtypeStruct((_ROWS, _COLS), jnp.float32),
        ],
        mesh=mesh,
        scratch_types=[
            pltpu.VMEM((_NBUF, _RCH, _COLS), jnp.float32),
            pltpu.VMEM((_NBUF, _RCH, _COLS), jnp.int32),
            pltpu.VMEM((_NBUF, _RCH, _COLS), jnp.float32),
            pltpu.VMEM((4, 16), jnp.float32),
            pltpu.SemaphoreType.DMA((_NBUF,)),
            pltpu.SemaphoreType.DMA((_NBUF,)),
            pltpu.SemaphoreType.DMA((_NBUF,)),
        ],
    )(_sc_body)
    return call(z, params)


def kernel(z, boundaries):
    b0 = boundaries[0]
    step = (boundaries[_LEVELS - 1] - b0) * (1.0 / (_LEVELS - 1))
    scale = 1.0 / step
    shift = -b0 * scale + 0.5
    params = jnp.stack([
        jnp.full((16,), scale, jnp.float32),
        jnp.full((16,), shift, jnp.float32),
        jnp.full((16,), step, jnp.float32),
        jnp.full((16,), b0, jnp.float32),
    ])

    idx, zq = _sc_quantize(z, params)
    return zq, idx
